# Initial kernel scaffold; baseline (speedup 1.0000x reference)
import jax
import jax.numpy as jnp
from jax.experimental import pallas as pl


def kernel(scores, bbox_frame, im_info):
    raise NotImplementedError("write your pallas kernel here")


# speedup vs baseline: 18.3465x; 18.3465x over previous
import functools

import numpy as np
import jax
import jax.numpy as jnp
from jax import lax
from jax.experimental import pallas as pl
from jax.experimental.pallas import tpu as pltpu
from jax.experimental.pallas import tpu_sc as plsc

_FEAT_STRIDE = 16
_SCALES = [4.0, 8.0, 16.0]
_RATIOS = [0.5, 1.0, 2.0]
_TIME_DIM = [8, 4]
_SAMPLE_DURATION = 8
_K = 2000
_B = 4
_N = 55296
_NW = 8
_CHUNK = _N // _NW
_NVREG = _CHUNK // 16
_CMAX = 4096
_CBUF = 8192
_RPW = 256
_OUTW = 40


def _gen_base_anchors(base_size=16):
    base = np.array([1.0, 1.0, base_size, base_size], dtype=np.float64) - 1.0
    w = base[2] - base[0] + 1.0
    h = base[3] - base[1] + 1.0
    xc = base[0] + 0.5 * (w - 1.0)
    yc = base[1] + 0.5 * (h - 1.0)
    rows = []
    for r in _RATIOS:
        size_r = (w * h) / r
        ws = np.round(np.sqrt(size_r))
        hs = np.round(ws * r)
        for s in _SCALES:
            ws2 = ws * s
            hs2 = hs * s
            rows.append([xc - 0.5 * (ws2 - 1.0), yc - 0.5 * (hs2 - 1.0),
                         xc + 0.5 * (ws2 - 1.0), yc + 0.5 * (hs2 - 1.0)])
    return np.array(rows, dtype=np.float32)


def _anchors_table(feat_h=32, feat_w=32):
    anchors = _gen_base_anchors()
    A = anchors.shape[0]
    shift_x = np.arange(0, feat_w) * _FEAT_STRIDE
    shift_y = np.arange(0, feat_h) * _FEAT_STRIDE
    sx, sy = np.meshgrid(shift_x, shift_y)
    shifts = np.vstack((sx.ravel(), sy.ravel(), sx.ravel(), sy.ravel()))
    shifts = shifts.transpose().astype(np.float32)
    Kp = shifts.shape[0]
    anc = anchors[None, :, :] + shifts[:, None, :]
    anc = anc.reshape(Kp * A, 4)
    parts = []
    for t in _TIME_DIM:
        for j in range(0, _SAMPLE_DURATION - t + 1):
            a = np.zeros((_SAMPLE_DURATION, Kp * A, 4), dtype=np.float32)
            a[j:j + t] = anc
            parts.append(a.transpose(1, 0, 2))
    out = np.concatenate(parts, 0)
    return out.reshape(_N, _SAMPLE_DURATION * 4)


_ANCHORS = _anchors_table()


def _sc_body(scores_hbm, bf_hbm, anch_hbm, imf_hbm, out_hbm,
             sf32, keys, hist, tmph, bins, scal, cnts,
             cand_u, cand_n, cand_u2, cand_n2,
             topv, topn, baseb, idxb, dsoa, arow, asoa, outb, imv,
             sh_hist, sh_cnt, sh_bc, sh_cu, sh_cn, sh_tv, sh_tn,
             sem, gsem):
    c = lax.axis_index("c")
    s = lax.axis_index("s")
    bslot = s // 8
    w8 = s % 8
    b = c * 2 + bslot
    lane = lax.iota(jnp.int32, 16)
    ones = jnp.full((16,), 1, jnp.int32)
    u32 = jnp.uint32

    pltpu.sync_copy(
        scores_hbm.at[pl.ds(
            pl.multiple_of(b * 110592 + 55296 + w8 * _CHUNK, 8), _CHUNK)],
        sf32)
    pltpu.sync_copy(imf_hbm, imv)

    def _zero_hist(t, _):
        hist[pl.ds(t * 16, 16)] = jnp.zeros((16,), jnp.int32)
        return 0
    lax.fori_loop(0, 256, _zero_hist, 0)

    def _init_cand(t, _):
        cand_u[pl.ds(t * 16, 16)] = jnp.full((16,), -1, jnp.int32)
        cand_n[pl.ds(t * 16, 16)] = jnp.zeros((16,), jnp.int32)
        return 0
    lax.fori_loop(0, _CBUF // 16, _init_cand, 0)

    def _keys_hist(t, _):
        f = sf32[pl.ds(t * 16, 16)]
        x = lax.bitcast_convert_type(f, u32)
        key = x ^ (((x >> 31) * u32(0x7FFFFFFF)) + u32(0x80000000))
        keys[pl.ds(t * 16, 16)] = key
        dig = (key >> 24).astype(jnp.int32)
        plsc.addupdate_scatter(hist, [(dig << 4) | lane], ones)
        return 0
    lax.fori_loop(0, _NVREG, _keys_hist, 0)
    pltpu.sync_copy(hist, sh_hist.at[bslot, w8])
    plsc.subcore_barrier()

    def _merge_hists():
        def _acc(ww, _):
            pltpu.sync_copy(sh_hist.at[bslot, ww], tmph)
            def _add(t, _):
                hist[pl.ds(t * 16, 16)] = (hist[pl.ds(t * 16, 16)]
                                           + tmph[pl.ds(t * 16, 16)])
                return 0
            lax.fori_loop(0, 256, _add, 0)
            return 0
        lax.fori_loop(1, 8, _acc, 0)
        def _binsum(d, _):
            bins[d] = jnp.sum(hist[pl.ds(d * 16, 16)])
            return 0
        lax.fori_loop(0, 256, _binsum, 0)

    def _scan_bins(target):
        def _scan(t, carry):
            cum, dd, above = carry
            d = 255 - t
            cnt = bins[d]
            found = (dd < 0) & (cum + cnt >= target)
            dd = jnp.where(found, d, dd)
            above = jnp.where(found, cum, above)
            return (cum + cnt, dd, above)
        _, d1, above = lax.fori_loop(
            0, 256, _scan, (jnp.int32(0), jnp.int32(-1), jnp.int32(0)))
        return d1, above

    @pl.when(w8 == 0)
    def _():
        _merge_hists()
        d1, above = _scan_bins(jnp.int32(_K))
        scal[pl.ds(0, 16)] = jnp.where(lane == 0, d1,
                                       jnp.where(lane == 1, above, 0))
        pltpu.sync_copy(scal, sh_bc.at[bslot])
    plsc.subcore_barrier()

    pltpu.sync_copy(sh_bc.at[bslot], scal)
    _bcv = scal[pl.ds(0, 16)]
    d1 = _bcv[0]
    above1 = _bcv[1]
    lax.fori_loop(0, 256, _zero_hist, 0)
    d1u = d1.astype(u32)

    def _hist2(t, _):
        key = keys[pl.ds(t * 16, 16)]
        m = (key >> 24) == d1u
        dig = ((key >> 16) & u32(0xFF)).astype(jnp.int32)
        plsc.addupdate_scatter(hist, [(dig << 4) | lane], ones, mask=m)
        return 0
    lax.fori_loop(0, _NVREG, _hist2, 0)
    pltpu.sync_copy(hist, sh_hist.at[bslot, w8])
    plsc.subcore_barrier()

    @pl.when(w8 == 0)
    def _():
        _merge_hists()
        d2, _ = _scan_bins(_K - above1)
        scal[pl.ds(0, 16)] = lane * 0 + ((d1 << 8) | d2)
        pltpu.sync_copy(scal, sh_bc.at[bslot])
    plsc.subcore_barrier()

    pltpu.sync_copy(sh_bc.at[bslot], scal)
    t16u = scal[pl.ds(0, 16)][0].astype(u32)

    def _collect(t, off):
        key = keys[pl.ds(t * 16, 16)]
        m = (key >> 16) >= t16u
        mi = m.astype(jnp.int32)
        pos = off + plsc.cumsum(mi) - 1
        plsc.store_scatter(cand_u, [pos], lax.bitcast_convert_type(~key, jnp.int32), mask=m)
        j = w8 * _CHUNK + t * 16 + lane
        a = j >> 10
        rem = j & 1023
        n = (rem >> 5) * 1728 + (rem & 31) * 54 + a
        plsc.store_scatter(cand_n, [pos], n, mask=m)
        return off + jnp.sum(mi)
    cw = lax.fori_loop(0, _NVREG, _collect, jnp.int32(0))
    scal[pl.ds(0, 16)] = lane * 0 + cw
    pltpu.sync_copy(scal, sh_cnt.at[bslot, w8])
    pltpu.sync_copy(cand_u.at[pl.ds(0, _CMAX)], sh_cu.at[bslot, w8])
    pltpu.sync_copy(cand_n.at[pl.ds(0, _CMAX)], sh_cn.at[bslot, w8])
    plsc.subcore_barrier()

    @pl.when(w8 == 0)
    def _():
        pltpu.sync_copy(sh_cnt.at[bslot], cnts)

        def _pack(ww, off):
            off = pl.multiple_of(jnp.minimum(off, _CMAX), 8)
            pltpu.sync_copy(sh_cu.at[bslot, ww], cand_u.at[pl.ds(off, _CMAX)])
            pltpu.sync_copy(sh_cn.at[bslot, ww], cand_n.at[pl.ds(off, _CMAX)])
            cww = cnts[ww, pl.ds(0, 16)][0]
            return off + ((cww + 7) & ~7)
        ctot = lax.fori_loop(0, 8, _pack, jnp.int32(0))
        seg = (ctot + 15) // 16

        def _radix_pass(src_u, src_n, dst_u, dst_n, shift, from_n):
            lax.fori_loop(0, 256, _zero_hist, 0)

            def _h(t, _):
                idx = lane * seg + t
                if from_n:
                    d = (plsc.load_gather(src_n, [idx]) >> shift) & 255
                else:
                    v = plsc.load_gather(src_u, [idx])
                    d = (v >> shift) & 255
                plsc.addupdate_scatter(hist, [(d << 4) | lane], ones)
                return 0
            lax.fori_loop(0, seg, _h, 0)

            def _prefix(d, run):
                vec = hist[pl.ds(d * 16, 16)]
                cs = plsc.cumsum(vec)
                hist[pl.ds(d * 16, 16)] = cs - vec + run
                return run + jnp.sum(vec)
            lax.fori_loop(0, 256, _prefix, jnp.int32(0))

            def _p(t, _):
                idx = lane * seg + t
                v = plsc.load_gather(src_u, [idx])
                nn = plsc.load_gather(src_n, [idx])
                if from_n:
                    d = (nn >> shift) & 255
                else:
                    d = (v >> shift) & 255
                cls = (d << 4) | lane
                pos = plsc.load_gather(hist, [cls])
                plsc.store_scatter(dst_u, [pos], v)
                plsc.store_scatter(dst_n, [pos], nn)
                plsc.addupdate_scatter(hist, [cls], ones)
                return 0
            lax.fori_loop(0, seg, _p, 0)

        _radix_pass(cand_u, cand_n, cand_u2, cand_n2, 0, True)
        _radix_pass(cand_u2, cand_n2, cand_u, cand_n, 8, True)
        _radix_pass(cand_u, cand_n, cand_u2, cand_n2, 0, False)
        _radix_pass(cand_u2, cand_n2, cand_u, cand_n, 8, False)
        _radix_pass(cand_u, cand_n, cand_u2, cand_n2, 16, False)
        _radix_pass(cand_u2, cand_n2, cand_u, cand_n, 24, False)

        pltpu.sync_copy(cand_u.at[pl.ds(0, 2048)], sh_tv.at[bslot])
        pltpu.sync_copy(cand_n.at[pl.ds(0, 2048)], sh_tn.at[bslot])
    plsc.subcore_barrier()

    r0 = pl.multiple_of(w8 * _RPW, 8)
    pltpu.sync_copy(sh_tv.at[bslot, pl.ds(r0, _RPW)], topv)
    pltpu.sync_copy(sh_tn.at[bslot, pl.ds(r0, _RPW)], topn)

    boff = b * 1769472

    def _bases(cc, _):
        n = topn[pl.ds(cc * 16, 16)]
        base = (n % 54) * 32768 + (n // 1728) * 32 + ((n // 54) % 32)
        baseb[pl.ds(cc * 16, 16)] = base + boff
        return 0
    lax.fori_loop(0, 16, _bases, 0)

    def _fill_idx(t, _):
        k = t >> 4
        cc = t & 15
        bv = baseb[pl.ds(cc * 16, 16)]
        idxb[pl.ds(k * _RPW + cc * 16, 16)] = bv + k * 1024
        return 0
    lax.fori_loop(0, 512, _fill_idx, 0)

    def _gather8(g, _):
        descs = []
        for i in range(8):
            jj = g * 8 + i
            descs.append(pltpu.async_copy(
                bf_hbm.at[idxb.at[pl.ds(jj * 128, 128)]],
                dsoa.at[pl.ds(jj * 128, 128)], gsem))
        for dsc in descs:
            dsc.wait()
        return 0
    lax.fori_loop(0, 8, _gather8, 0)

    for i in range(2):
        pltpu.async_copy(
            anch_hbm.at[topn.at[pl.ds(i * 128, 128)]],
            arow.at[pl.ds(i * 128, 128), :], sem).wait()

    def _tr(t, _):
        k = t >> 4
        cc = t & 15
        row = cc * 16 + lane
        v = plsc.load_gather(arow, [row, jnp.full((16,), k, jnp.int32)])
        asoa[pl.ds(k * _RPW + cc * 16, 16)] = v
        return 0
    lax.fori_loop(0, 512, _tr, 0)

    _imvec = imv[pl.ds(0, 16)]
    wmax = jnp.sum(jnp.where(lane == b * 4 + 1, _imvec, 0.0)) - 1.0
    hmax = jnp.sum(jnp.where(lane == b * 4 + 0, _imvec, 0.0)) - 1.0

    def _transform(cc, _):
        cvec = (cc * 16 + lane) * _OUTW
        for f in range(8):
            q = 4 * f * _RPW + cc * 16
            ax1 = asoa[pl.ds(q, 16)]
            ay1 = asoa[pl.ds(q + _RPW, 16)]
            ax2 = asoa[pl.ds(q + 2 * _RPW, 16)]
            ay2 = asoa[pl.ds(q + 3 * _RPW, 16)]
            dx = dsoa[pl.ds(q, 16)]
            dy = dsoa[pl.ds(q + _RPW, 16)]
            dw = dsoa[pl.ds(q + 2 * _RPW, 16)]
            dh = dsoa[pl.ds(q + 3 * _RPW, 16)]
            ww = ax2 - ax1 + 1.0
            hh = ay2 - ay1 + 1.0
            cx = ax1 + 0.5 * ww
            cy = ay1 + 0.5 * hh
            px = dx * ww + cx
            py = dy * hh + cy
            pw = jnp.exp(dw) * ww
            ph = jnp.exp(dh) * hh
            x1 = jnp.clip(px - 0.5 * pw, 0.0, wmax)
            y1 = jnp.clip(py - 0.5 * ph, 0.0, hmax)
            x2 = jnp.clip(px + 0.5 * pw, 0.0, wmax)
            y2 = jnp.clip(py + 0.5 * ph, 0.0, hmax)
            plsc.store_scatter(outb, [cvec + (1 + 4 * f)], x1)
            plsc.store_scatter(outb, [cvec + (2 + 4 * f)], y1)
            plsc.store_scatter(outb, [cvec + (3 + 4 * f)], x2)
            plsc.store_scatter(outb, [cvec + (4 + 4 * f)], y2)
        return 0
    lax.fori_loop(0, 16, _transform, 0)

    bf32 = b.astype(jnp.float32)

    def _meta_cols(cc, _):
        cvec = (cc * 16 + lane) * _OUTW
        v = topv[pl.ds(cc * 16, 16)]
        key = lax.bitcast_convert_type(~v, u32)
        sb = key ^ ((((~key) >> 31) * u32(0x7FFFFFFF)) + u32(0x80000000))
        plsc.store_scatter(outb, [cvec + 33], lax.bitcast_convert_type(sb, jnp.float32))
        plsc.store_scatter(outb, [cvec], jnp.full((16,), 1.0, jnp.float32) * bf32)
        return 0
    lax.fori_loop(0, 16, _meta_cols, 0)

    pltpu.sync_copy(outb, out_hbm.at[pl.ds(
        pl.multiple_of((b * 2048 + r0) * _OUTW, 8), _RPW * _OUTW)])


@jax.jit
def kernel(scores, bbox_frame, im_info):
    f32 = jnp.float32
    i32 = jnp.int32
    u32 = jnp.uint32
    mesh = plsc.VectorSubcoreMesh(core_axis_name="c", subcore_axis_name="s")
    run = functools.partial(
        pl.kernel,
        out_type=jax.ShapeDtypeStruct((_B * 2048 * _OUTW,), f32),
        mesh=mesh,
        compiler_params=pltpu.CompilerParams(needs_layout_passes=False,
                                             use_tc_tiling_on_sc=False),
        scratch_types=[
            pltpu.VMEM((_CHUNK,), f32),
            pltpu.VMEM((_CHUNK,), u32),
            pltpu.VMEM((4096,), i32),
            pltpu.VMEM((4096,), i32),
            pltpu.SMEM((256,), i32),
            pltpu.VMEM((16,), i32),
            pltpu.VMEM((8, 16), i32),
            pltpu.VMEM((_CBUF,), i32),
            pltpu.VMEM((_CBUF,), i32),
            pltpu.VMEM((_CBUF,), i32),
            pltpu.VMEM((_CBUF,), i32),
            pltpu.VMEM((_RPW,), i32),
            pltpu.VMEM((_RPW,), i32),
            pltpu.VMEM((_RPW,), i32),
            pltpu.VMEM((32 * _RPW,), i32),
            pltpu.VMEM((32 * _RPW,), f32),
            pltpu.VMEM((_RPW, 32), f32),
            pltpu.VMEM((32 * _RPW,), f32),
            pltpu.VMEM((_RPW * _OUTW,), f32),
            pltpu.VMEM((16,), f32),
            pltpu.VMEM_SHARED((2, 8, 4096), i32),
            pltpu.VMEM_SHARED((2, 8, 16), i32),
            pltpu.VMEM_SHARED((2, 16), i32),
            pltpu.VMEM_SHARED((2, 8, _CMAX), i32),
            pltpu.VMEM_SHARED((2, 8, _CMAX), i32),
            pltpu.VMEM_SHARED((2, 2048), i32),
            pltpu.VMEM_SHARED((2, 2048), i32),
            pltpu.SemaphoreType.DMA,
            pltpu.SemaphoreType.DMA,
        ],
    )(_sc_body)

    imf = jnp.pad(im_info, ((0, 0), (0, 1))).reshape(-1)
    out = run(scores.reshape(-1), bbox_frame.reshape(-1),
              jnp.asarray(_ANCHORS), imf)
    return out.reshape(_B, 2048, _OUTW)[:, :_K, :34]


# speedup vs baseline: 18.3580x; 1.0006x over previous
import functools

import numpy as np
import jax
import jax.numpy as jnp
from jax import lax
from jax.experimental import pallas as pl
from jax.experimental.pallas import tpu as pltpu
from jax.experimental.pallas import tpu_sc as plsc

_FEAT_STRIDE = 16
_SCALES = [4.0, 8.0, 16.0]
_RATIOS = [0.5, 1.0, 2.0]
_TIME_DIM = [8, 4]
_SAMPLE_DURATION = 8
_K = 2000
_B = 4
_N = 55296
_NW = 8
_CHUNK = _N // _NW
_NVREG = _CHUNK // 16
_CMAX = 4096
_CBUF = 8192
_RPW = 256
_OUTW = 40


def _gen_base_anchors(base_size=16):
    base = np.array([1.0, 1.0, base_size, base_size], dtype=np.float64) - 1.0
    w = base[2] - base[0] + 1.0
    h = base[3] - base[1] + 1.0
    xc = base[0] + 0.5 * (w - 1.0)
    yc = base[1] + 0.5 * (h - 1.0)
    rows = []
    for r in _RATIOS:
        size_r = (w * h) / r
        ws = np.round(np.sqrt(size_r))
        hs = np.round(ws * r)
        for s in _SCALES:
            ws2 = ws * s
            hs2 = hs * s
            rows.append([xc - 0.5 * (ws2 - 1.0), yc - 0.5 * (hs2 - 1.0),
                         xc + 0.5 * (ws2 - 1.0), yc + 0.5 * (hs2 - 1.0)])
    return np.array(rows, dtype=np.float32)


def _anchors_table(feat_h=32, feat_w=32):
    anchors = _gen_base_anchors()
    A = anchors.shape[0]
    shift_x = np.arange(0, feat_w) * _FEAT_STRIDE
    shift_y = np.arange(0, feat_h) * _FEAT_STRIDE
    sx, sy = np.meshgrid(shift_x, shift_y)
    shifts = np.vstack((sx.ravel(), sy.ravel(), sx.ravel(), sy.ravel()))
    shifts = shifts.transpose().astype(np.float32)
    Kp = shifts.shape[0]
    anc = anchors[None, :, :] + shifts[:, None, :]
    anc = anc.reshape(Kp * A, 4)
    parts = []
    for t in _TIME_DIM:
        for j in range(0, _SAMPLE_DURATION - t + 1):
            a = np.zeros((_SAMPLE_DURATION, Kp * A, 4), dtype=np.float32)
            a[j:j + t] = anc
            parts.append(a.transpose(1, 0, 2))
    out = np.concatenate(parts, 0)
    return out.reshape(_N, _SAMPLE_DURATION * 4)


_ANCHORS = _anchors_table()


def _sc_body(scores_hbm, bf_hbm, anch_hbm, imf_hbm, out_hbm,
             sf32, keys, hist, tmph, bins, scal, cnts,
             cand_u, cand_n, cand_u2, cand_n2,
             topv, topn, baseb, idxb, dsoa, arow, asoa, outb, imv,
             sh_hist, sh_cnt, sh_bc, sh_cu, sh_cn, sh_tv, sh_tn,
             sem, gsem):
    c = lax.axis_index("c")
    s = lax.axis_index("s")
    bslot = s // 8
    w8 = s % 8
    b = c * 2 + bslot
    lane = lax.iota(jnp.int32, 16)
    ones = jnp.full((16,), 1, jnp.int32)
    u32 = jnp.uint32

    pltpu.sync_copy(
        scores_hbm.at[pl.ds(
            pl.multiple_of(b * 110592 + 55296 + w8 * _CHUNK, 8), _CHUNK)],
        sf32)
    pltpu.sync_copy(imf_hbm, imv)

    def _zero_hist(t, _):
        hist[pl.ds(t * 16, 16)] = jnp.zeros((16,), jnp.int32)
        return 0
    lax.fori_loop(0, 256, _zero_hist, 0)

    def _init_cand(t, _):
        cand_u[pl.ds(t * 16, 16)] = jnp.full((16,), -1, jnp.int32)
        cand_n[pl.ds(t * 16, 16)] = jnp.zeros((16,), jnp.int32)
        return 0
    lax.fori_loop(0, _CBUF // 16, _init_cand, 0)

    def _keys_hist(t, _):
        f = sf32[pl.ds(t * 16, 16)]
        x = lax.bitcast_convert_type(f, u32)
        key = x ^ (((x >> 31) * u32(0x7FFFFFFF)) + u32(0x80000000))
        keys[pl.ds(t * 16, 16)] = key
        dig = (key >> 24).astype(jnp.int32)
        plsc.addupdate_scatter(hist, [(dig << 4) | lane], ones)
        return 0
    with jax.named_scope("p1_keys"):
        lax.fori_loop(0, _NVREG, _keys_hist, 0)
    pltpu.sync_copy(hist, sh_hist.at[bslot, w8])
    plsc.subcore_barrier()

    def _merge_hists():
        def _acc(ww, _):
            pltpu.sync_copy(sh_hist.at[bslot, ww], tmph)
            def _add(t, _):
                hist[pl.ds(t * 16, 16)] = (hist[pl.ds(t * 16, 16)]
                                           + tmph[pl.ds(t * 16, 16)])
                return 0
            lax.fori_loop(0, 256, _add, 0)
            return 0
        lax.fori_loop(1, 8, _acc, 0)
        def _binsum(d, _):
            bins[d] = jnp.sum(hist[pl.ds(d * 16, 16)])
            return 0
        lax.fori_loop(0, 256, _binsum, 0)

    def _scan_bins(target):
        def _scan(t, carry):
            cum, dd, above = carry
            d = 255 - t
            cnt = bins[d]
            found = (dd < 0) & (cum + cnt >= target)
            dd = jnp.where(found, d, dd)
            above = jnp.where(found, cum, above)
            return (cum + cnt, dd, above)
        _, d1, above = lax.fori_loop(
            0, 256, _scan, (jnp.int32(0), jnp.int32(-1), jnp.int32(0)))
        return d1, above

    @pl.when(w8 == 0)
    def _():
        with jax.named_scope("p2_merge"):
            _merge_hists()
        d1, above = _scan_bins(jnp.int32(_K))
        scal[pl.ds(0, 16)] = jnp.where(lane == 0, d1,
                                       jnp.where(lane == 1, above, 0))
        pltpu.sync_copy(scal, sh_bc.at[bslot])
    plsc.subcore_barrier()

    pltpu.sync_copy(sh_bc.at[bslot], scal)
    _bcv = scal[pl.ds(0, 16)]
    d1 = _bcv[0]
    above1 = _bcv[1]
    lax.fori_loop(0, 256, _zero_hist, 0)
    d1u = d1.astype(u32)

    def _hist2(t, _):
        key = keys[pl.ds(t * 16, 16)]
        m = (key >> 24) == d1u
        dig = ((key >> 16) & u32(0xFF)).astype(jnp.int32)
        plsc.addupdate_scatter(hist, [(dig << 4) | lane], ones, mask=m)
        return 0
    with jax.named_scope("p3_hist2"):
        lax.fori_loop(0, _NVREG, _hist2, 0)
    pltpu.sync_copy(hist, sh_hist.at[bslot, w8])
    plsc.subcore_barrier()

    @pl.when(w8 == 0)
    def _():
        _merge_hists()
        d2, _ = _scan_bins(_K - above1)
        scal[pl.ds(0, 16)] = lane * 0 + ((d1 << 8) | d2)
        pltpu.sync_copy(scal, sh_bc.at[bslot])
    plsc.subcore_barrier()

    pltpu.sync_copy(sh_bc.at[bslot], scal)
    t16u = scal[pl.ds(0, 16)][0].astype(u32)

    def _collect(t, off):
        key = keys[pl.ds(t * 16, 16)]
        m = (key >> 16) >= t16u
        mi = m.astype(jnp.int32)
        pos = off + plsc.cumsum(mi) - 1
        plsc.store_scatter(cand_u, [pos], lax.bitcast_convert_type(~key, jnp.int32), mask=m)
        j = w8 * _CHUNK + t * 16 + lane
        a = j >> 10
        rem = j & 1023
        n = (rem >> 5) * 1728 + (rem & 31) * 54 + a
        plsc.store_scatter(cand_n, [pos], n, mask=m)
        return off + jnp.sum(mi)
    with jax.named_scope("p4_collect"):
        cw = lax.fori_loop(0, _NVREG, _collect, jnp.int32(0))
    scal[pl.ds(0, 16)] = lane * 0 + cw
    pltpu.sync_copy(scal, sh_cnt.at[bslot, w8])
    pltpu.sync_copy(cand_u.at[pl.ds(0, _CMAX)], sh_cu.at[bslot, w8])
    pltpu.sync_copy(cand_n.at[pl.ds(0, _CMAX)], sh_cn.at[bslot, w8])
    plsc.subcore_barrier()

    @pl.when(w8 == 0)
    def _():
        pltpu.sync_copy(sh_cnt.at[bslot], cnts)

        def _pack(ww, off):
            off = pl.multiple_of(jnp.minimum(off, _CMAX), 8)
            pltpu.sync_copy(sh_cu.at[bslot, ww], cand_u.at[pl.ds(off, _CMAX)])
            pltpu.sync_copy(sh_cn.at[bslot, ww], cand_n.at[pl.ds(off, _CMAX)])
            cww = cnts[ww, pl.ds(0, 16)][0]
            return off + ((cww + 7) & ~7)
        ctot = lax.fori_loop(0, 8, _pack, jnp.int32(0))
        seg = (ctot + 15) // 16

        def _radix_pass(src_u, src_n, dst_u, dst_n, shift, from_n):
            lax.fori_loop(0, 256, _zero_hist, 0)

            def _h(t, _):
                idx = lane * seg + t
                if from_n:
                    d = (plsc.load_gather(src_n, [idx]) >> shift) & 255
                else:
                    v = plsc.load_gather(src_u, [idx])
                    d = (v >> shift) & 255
                plsc.addupdate_scatter(hist, [(d << 4) | lane], ones)
                return 0
            lax.fori_loop(0, seg, _h, 0)

            def _prefix(d, run):
                vec = hist[pl.ds(d * 16, 16)]
                cs = plsc.cumsum(vec)
                hist[pl.ds(d * 16, 16)] = cs - vec + run
                return run + jnp.sum(vec)
            lax.fori_loop(0, 256, _prefix, jnp.int32(0))

            def _p(t, _):
                idx = lane * seg + t
                v = plsc.load_gather(src_u, [idx])
                nn = plsc.load_gather(src_n, [idx])
                if from_n:
                    d = (nn >> shift) & 255
                else:
                    d = (v >> shift) & 255
                cls = (d << 4) | lane
                pos = plsc.load_gather(hist, [cls])
                plsc.store_scatter(dst_u, [pos], v)
                plsc.store_scatter(dst_n, [pos], nn)
                plsc.addupdate_scatter(hist, [cls], ones)
                return 0
            lax.fori_loop(0, seg, _p, 0)

        with jax.named_scope("p5_sort"):
            _radix_pass(cand_u, cand_n, cand_u2, cand_n2, 0, True)
            _radix_pass(cand_u2, cand_n2, cand_u, cand_n, 8, True)
            _radix_pass(cand_u, cand_n, cand_u2, cand_n2, 0, False)
            _radix_pass(cand_u2, cand_n2, cand_u, cand_n, 8, False)
            _radix_pass(cand_u, cand_n, cand_u2, cand_n2, 16, False)
            _radix_pass(cand_u2, cand_n2, cand_u, cand_n, 24, False)

        pltpu.sync_copy(cand_u.at[pl.ds(0, 2048)], sh_tv.at[bslot])
        pltpu.sync_copy(cand_n.at[pl.ds(0, 2048)], sh_tn.at[bslot])
    plsc.subcore_barrier()

    r0 = pl.multiple_of(w8 * _RPW, 8)
    pltpu.sync_copy(sh_tv.at[bslot, pl.ds(r0, _RPW)], topv)
    pltpu.sync_copy(sh_tn.at[bslot, pl.ds(r0, _RPW)], topn)

    boff = b * 1769472

    def _bases(cc, _):
        n = topn[pl.ds(cc * 16, 16)]
        base = (n % 54) * 32768 + (n // 1728) * 32 + ((n // 54) % 32)
        baseb[pl.ds(cc * 16, 16)] = base + boff
        return 0
    lax.fori_loop(0, 16, _bases, 0)

    def _fill_idx(t, _):
        k = t >> 4
        cc = t & 15
        bv = baseb[pl.ds(cc * 16, 16)]
        idxb[pl.ds(k * _RPW + cc * 16, 16)] = bv + k * 1024
        return 0
    lax.fori_loop(0, 512, _fill_idx, 0)

    def _gather8(g, _):
        descs = []
        for i in range(8):
            jj = g * 8 + i
            descs.append(pltpu.async_copy(
                bf_hbm.at[idxb.at[pl.ds(jj * 128, 128)]],
                dsoa.at[pl.ds(jj * 128, 128)], gsem))
        for dsc in descs:
            dsc.wait()
        return 0
    with jax.named_scope("p6_gather"):
        lax.fori_loop(0, 8, _gather8, 0)

    for i in range(2):
        pltpu.async_copy(
            anch_hbm.at[topn.at[pl.ds(i * 128, 128)]],
            arow.at[pl.ds(i * 128, 128), :], sem).wait()

    def _tr(t, _):
        k = t >> 4
        cc = t & 15
        row = cc * 16 + lane
        v = plsc.load_gather(arow, [row, jnp.full((16,), k, jnp.int32)])
        asoa[pl.ds(k * _RPW + cc * 16, 16)] = v
        return 0
    with jax.named_scope("p6_transpose"):
        lax.fori_loop(0, 512, _tr, 0)

    _imvec = imv[pl.ds(0, 16)]
    wmax = jnp.sum(jnp.where(lane == b * 4 + 1, _imvec, 0.0)) - 1.0
    hmax = jnp.sum(jnp.where(lane == b * 4 + 0, _imvec, 0.0)) - 1.0

    def _transform(cc, _):
        cvec = (cc * 16 + lane) * _OUTW
        for f in range(8):
            q = 4 * f * _RPW + cc * 16
            ax1 = asoa[pl.ds(q, 16)]
            ay1 = asoa[pl.ds(q + _RPW, 16)]
            ax2 = asoa[pl.ds(q + 2 * _RPW, 16)]
            ay2 = asoa[pl.ds(q + 3 * _RPW, 16)]
            dx = dsoa[pl.ds(q, 16)]
            dy = dsoa[pl.ds(q + _RPW, 16)]
            dw = dsoa[pl.ds(q + 2 * _RPW, 16)]
            dh = dsoa[pl.ds(q + 3 * _RPW, 16)]
            ww = ax2 - ax1 + 1.0
            hh = ay2 - ay1 + 1.0
            cx = ax1 + 0.5 * ww
            cy = ay1 + 0.5 * hh
            px = dx * ww + cx
            py = dy * hh + cy
            pw = jnp.exp(dw) * ww
            ph = jnp.exp(dh) * hh
            x1 = jnp.clip(px - 0.5 * pw, 0.0, wmax)
            y1 = jnp.clip(py - 0.5 * ph, 0.0, hmax)
            x2 = jnp.clip(px + 0.5 * pw, 0.0, wmax)
            y2 = jnp.clip(py + 0.5 * ph, 0.0, hmax)
            plsc.store_scatter(outb, [cvec + (1 + 4 * f)], x1)
            plsc.store_scatter(outb, [cvec + (2 + 4 * f)], y1)
            plsc.store_scatter(outb, [cvec + (3 + 4 * f)], x2)
            plsc.store_scatter(outb, [cvec + (4 + 4 * f)], y2)
        return 0
    with jax.named_scope("p6_transform"):
        lax.fori_loop(0, 16, _transform, 0)

    bf32 = b.astype(jnp.float32)

    def _meta_cols(cc, _):
        cvec = (cc * 16 + lane) * _OUTW
        v = topv[pl.ds(cc * 16, 16)]
        key = lax.bitcast_convert_type(~v, u32)
        sb = key ^ ((((~key) >> 31) * u32(0x7FFFFFFF)) + u32(0x80000000))
        plsc.store_scatter(outb, [cvec + 33], lax.bitcast_convert_type(sb, jnp.float32))
        plsc.store_scatter(outb, [cvec], jnp.full((16,), 1.0, jnp.float32) * bf32)
        return 0
    lax.fori_loop(0, 16, _meta_cols, 0)

    pltpu.sync_copy(outb, out_hbm.at[pl.ds(
        pl.multiple_of((b * 2048 + r0) * _OUTW, 8), _RPW * _OUTW)])


@jax.jit
def kernel(scores, bbox_frame, im_info):
    f32 = jnp.float32
    i32 = jnp.int32
    u32 = jnp.uint32
    mesh = plsc.VectorSubcoreMesh(core_axis_name="c", subcore_axis_name="s")
    run = functools.partial(
        pl.kernel,
        out_type=jax.ShapeDtypeStruct((_B * 2048 * _OUTW,), f32),
        mesh=mesh,
        compiler_params=pltpu.CompilerParams(needs_layout_passes=False,
                                             use_tc_tiling_on_sc=False),
        scratch_types=[
            pltpu.VMEM((_CHUNK,), f32),
            pltpu.VMEM((_CHUNK,), u32),
            pltpu.VMEM((4096,), i32),
            pltpu.VMEM((4096,), i32),
            pltpu.SMEM((256,), i32),
            pltpu.VMEM((16,), i32),
            pltpu.VMEM((8, 16), i32),
            pltpu.VMEM((_CBUF,), i32),
            pltpu.VMEM((_CBUF,), i32),
            pltpu.VMEM((_CBUF,), i32),
            pltpu.VMEM((_CBUF,), i32),
            pltpu.VMEM((_RPW,), i32),
            pltpu.VMEM((_RPW,), i32),
            pltpu.VMEM((_RPW,), i32),
            pltpu.VMEM((32 * _RPW,), i32),
            pltpu.VMEM((32 * _RPW,), f32),
            pltpu.VMEM((_RPW, 32), f32),
            pltpu.VMEM((32 * _RPW,), f32),
            pltpu.VMEM((_RPW * _OUTW,), f32),
            pltpu.VMEM((16,), f32),
            pltpu.VMEM_SHARED((2, 8, 4096), i32),
            pltpu.VMEM_SHARED((2, 8, 16), i32),
            pltpu.VMEM_SHARED((2, 16), i32),
            pltpu.VMEM_SHARED((2, 8, _CMAX), i32),
            pltpu.VMEM_SHARED((2, 8, _CMAX), i32),
            pltpu.VMEM_SHARED((2, 2048), i32),
            pltpu.VMEM_SHARED((2, 2048), i32),
            pltpu.SemaphoreType.DMA,
            pltpu.SemaphoreType.DMA,
        ],
    )(_sc_body)

    imf = jnp.pad(im_info, ((0, 0), (0, 1))).reshape(-1)
    out = run(scores.reshape(-1), bbox_frame.reshape(-1),
              jnp.asarray(_ANCHORS), imf)
    return out.reshape(_B, 2048, _OUTW)[:, :_K, :34]


# speedup vs baseline: 22.1770x; 1.2080x over previous
import functools

import numpy as np
import jax
import jax.numpy as jnp
from jax import lax
from jax.experimental import pallas as pl
from jax.experimental.pallas import tpu as pltpu
from jax.experimental.pallas import tpu_sc as plsc

_FEAT_STRIDE = 16
_SCALES = [4.0, 8.0, 16.0]
_RATIOS = [0.5, 1.0, 2.0]
_TIME_DIM = [8, 4]
_SAMPLE_DURATION = 8
_K = 2000
_B = 4
_N = 55296
_NW = 8
_CHUNK = _N // _NW
_NVREG = _CHUNK // 16
_CMAX = 4096
_CBUF = 8192
_RPW = 256
_OUTW = 40


def _gen_base_anchors(base_size=16):
    base = np.array([1.0, 1.0, base_size, base_size], dtype=np.float64) - 1.0
    w = base[2] - base[0] + 1.0
    h = base[3] - base[1] + 1.0
    xc = base[0] + 0.5 * (w - 1.0)
    yc = base[1] + 0.5 * (h - 1.0)
    rows = []
    for r in _RATIOS:
        size_r = (w * h) / r
        ws = np.round(np.sqrt(size_r))
        hs = np.round(ws * r)
        for s in _SCALES:
            ws2 = ws * s
            hs2 = hs * s
            rows.append([xc - 0.5 * (ws2 - 1.0), yc - 0.5 * (hs2 - 1.0),
                         xc + 0.5 * (ws2 - 1.0), yc + 0.5 * (hs2 - 1.0)])
    return np.array(rows, dtype=np.float32)


def _anchors_table(feat_h=32, feat_w=32):
    anchors = _gen_base_anchors()
    A = anchors.shape[0]
    shift_x = np.arange(0, feat_w) * _FEAT_STRIDE
    shift_y = np.arange(0, feat_h) * _FEAT_STRIDE
    sx, sy = np.meshgrid(shift_x, shift_y)
    shifts = np.vstack((sx.ravel(), sy.ravel(), sx.ravel(), sy.ravel()))
    shifts = shifts.transpose().astype(np.float32)
    Kp = shifts.shape[0]
    anc = anchors[None, :, :] + shifts[:, None, :]
    anc = anc.reshape(Kp * A, 4)
    parts = []
    for t in _TIME_DIM:
        for j in range(0, _SAMPLE_DURATION - t + 1):
            a = np.zeros((_SAMPLE_DURATION, Kp * A, 4), dtype=np.float32)
            a[j:j + t] = anc
            parts.append(a.transpose(1, 0, 2))
    out = np.concatenate(parts, 0)
    return out.reshape(_N, _SAMPLE_DURATION * 4)


_ANCHORS = _anchors_table()


def _sc_body(scores_hbm, bf_hbm, anch_hbm, imf_hbm, out_hbm,
             sf32d, keys, hist, hist256, tmp256, scal, cnts,
             cand_u, cand_n, cand_u2, cand_n2,
             topv, topn, baseb, idxb, dsoa, arow, asoa, outb, imv,
             sh_hist, sh_cnt, sh_bc, sh_cu, sh_cn, sh_tv, sh_tn,
             sem, gsem):
    c = lax.axis_index("c")
    s = lax.axis_index("s")
    bslot = s // 8
    w8 = s % 8
    b = c * 2 + bslot
    lane = lax.iota(jnp.int32, 16)
    ones = jnp.full((16,), 1, jnp.int32)
    u32 = jnp.uint32
    i32 = jnp.int32

    pltpu.sync_copy(
        scores_hbm.at[b, pl.ds(54, 54), pl.ds(w8 * 4, 4), :], sf32d)
    pltpu.sync_copy(imf_hbm, imv)

    def _zero_hist(t, _):
        hist[pl.ds(t * 16, 16)] = jnp.zeros((16,), i32)
        return 0
    lax.fori_loop(0, 256, _zero_hist, 0, unroll=4)

    def _init_cand(t, _):
        cand_u[pl.ds(t * 16, 16)] = jnp.full((16,), -1, i32)
        cand_n[pl.ds(t * 16, 16)] = jnp.zeros((16,), i32)
        return 0
    lax.fori_loop(0, _CBUF // 16, _init_cand, 0, unroll=4)

    def _keys_hist(t, _):
        a = t // 8
        r = t - a * 8
        hh = r // 2
        q = r - hh * 2
        f = sf32d[a, hh, pl.ds(q * 16, 16)]
        x = lax.bitcast_convert_type(f, u32)
        key = x ^ (((x >> 31) * u32(0x7FFFFFFF)) + u32(0x80000000))
        pos = (hh * 32 + q * 16 + lane) * 54 + a
        plsc.store_scatter(keys, [pos], lax.bitcast_convert_type(key, i32))
        dig = (key >> 24).astype(i32)
        plsc.addupdate_scatter(hist, [lane * 256 + dig], ones)
        return 0
    with jax.named_scope("p1_keys"):
        lax.fori_loop(0, _NVREG, _keys_hist, 0, unroll=4)

    def _reduce_hist():
        def _red(t, _):
            acc = jnp.zeros((16,), i32)
            for l in range(16):
                acc = acc + hist[pl.ds(l * 256 + t * 16, 16)]
            hist256[pl.ds(t * 16, 16)] = acc
            return 0
        lax.fori_loop(0, 16, _red, 0)

    with jax.named_scope("p1_red"):
        _reduce_hist()
    pltpu.sync_copy(hist256, sh_hist.at[bslot, w8])
    plsc.subcore_barrier()

    def _merge256():
        def _acc(ww, _):
            pltpu.sync_copy(sh_hist.at[bslot, ww], tmp256)
            def _add(t, _):
                hist256[pl.ds(t * 16, 16)] = (hist256[pl.ds(t * 16, 16)]
                                              + tmp256[pl.ds(t * 16, 16)])
                return 0
            lax.fori_loop(0, 16, _add, 0, unroll=4)
            return 0
        lax.fori_loop(1, 8, _acc, 0)

    def _scan256(target):
        carry = jnp.int32(0)
        dd = jnp.int32(-1)
        for t in range(15, -1, -1):
            vec = hist256[pl.ds(t * 16, 16)]
            sfx = lax.rev(plsc.cumsum(lax.rev(vec, (0,))), (0,)) + carry
            tmp256[pl.ds(t * 16, 16)] = sfx
            carry = carry + jnp.sum(vec)
            dval = t * 16 + lane
            cnd = jnp.where(sfx >= target, dval, -1)
            dd = jnp.maximum(dd, jnp.max(cnd))
        above = jnp.int32(0)
        for t in range(16):
            dval = t * 16 + lane
            sel = dval == dd
            above = above + jnp.sum(
                jnp.where(sel, tmp256[pl.ds(t * 16, 16)]
                          - hist256[pl.ds(t * 16, 16)], 0))
        return dd, above

    @pl.when(w8 == 0)
    def _():
        with jax.named_scope("p2_merge"):
            _merge256()
            d1, above = _scan256(jnp.int32(_K))
        scal[pl.ds(0, 16)] = jnp.where(lane == 0, d1,
                                       jnp.where(lane == 1, above, 0))
        pltpu.sync_copy(scal, sh_bc.at[bslot])
    plsc.subcore_barrier()

    pltpu.sync_copy(sh_bc.at[bslot], scal)
    _bcv = scal[pl.ds(0, 16)]
    d1 = _bcv[0]
    above1 = _bcv[1]
    lax.fori_loop(0, 256, _zero_hist, 0, unroll=4)
    d1u = d1.astype(u32)

    def _hist2(t, _):
        key = lax.bitcast_convert_type(keys[pl.ds(t * 16, 16)], u32)
        m = (key >> 24) == d1u
        dig = ((key >> 16) & u32(0xFF)).astype(i32)
        plsc.addupdate_scatter(hist, [lane * 256 + dig], ones, mask=m)
        return 0
    with jax.named_scope("p3_hist2"):
        lax.fori_loop(0, _NVREG, _hist2, 0, unroll=4)
        _reduce_hist()
    pltpu.sync_copy(hist256, sh_hist.at[bslot, w8])
    plsc.subcore_barrier()

    @pl.when(w8 == 0)
    def _():
        with jax.named_scope("p3_merge"):
            _merge256()
            d2, _ = _scan256(_K - above1)
        scal[pl.ds(0, 16)] = lane * 0 + ((d1 << 8) | d2)
        pltpu.sync_copy(scal, sh_bc.at[bslot])
    plsc.subcore_barrier()

    pltpu.sync_copy(sh_bc.at[bslot], scal)
    t16u = scal[pl.ds(0, 16)][0].astype(u32)
    nbase = w8 * _CHUNK

    def _collect(t, off):
        key = lax.bitcast_convert_type(keys[pl.ds(t * 16, 16)], u32)
        m = (key >> 16) >= t16u
        mi = m.astype(i32)
        pos = off + plsc.cumsum(mi) - 1
        plsc.store_scatter(cand_u, [pos],
                           lax.bitcast_convert_type(~key, i32), mask=m)
        plsc.store_scatter(cand_n, [pos], nbase + t * 16 + lane, mask=m)
        return off + jnp.sum(mi)
    with jax.named_scope("p4_collect"):
        cw = lax.fori_loop(0, _NVREG, _collect, jnp.int32(0), unroll=4)
    scal[pl.ds(0, 16)] = lane * 0 + cw
    pltpu.sync_copy(scal, sh_cnt.at[bslot, w8])
    pltpu.sync_copy(cand_u.at[pl.ds(0, _CMAX)], sh_cu.at[bslot, w8])
    pltpu.sync_copy(cand_n.at[pl.ds(0, _CMAX)], sh_cn.at[bslot, w8])
    plsc.subcore_barrier()

    @pl.when(w8 == 0)
    def _():
        pltpu.sync_copy(sh_cnt.at[bslot], cnts)

        def _pack(ww, off):
            off = pl.multiple_of(jnp.minimum(off, _CMAX), 8)
            pltpu.sync_copy(sh_cu.at[bslot, ww], cand_u.at[pl.ds(off, _CMAX)])
            pltpu.sync_copy(sh_cn.at[bslot, ww], cand_n.at[pl.ds(off, _CMAX)])
            cww = cnts[ww, pl.ds(0, 16)][0]
            return off + ((cww + 7) & ~7)
        ctot = lax.fori_loop(0, 8, _pack, jnp.int32(0))
        seg = (ctot + 15) // 16

        def _radix_pass(src_u, src_n, dst_u, dst_n, shift):
            lax.fori_loop(0, 256, _zero_hist, 0, unroll=4)

            def _h(t, _):
                idx = lane * seg + t
                v = plsc.load_gather(src_u, [idx])
                d = (v >> shift) & 255
                plsc.addupdate_scatter(hist, [(d << 4) | lane], ones)
                return 0
            lax.fori_loop(0, seg, _h, 0)

            def _prefix(d, run):
                vec = hist[pl.ds(d * 16, 16)]
                cs = plsc.cumsum(vec)
                hist[pl.ds(d * 16, 16)] = cs - vec + run
                return run + jnp.sum(vec)
            lax.fori_loop(0, 256, _prefix, jnp.int32(0))

            def _p(t, _):
                idx = lane * seg + t
                v = plsc.load_gather(src_u, [idx])
                nn = plsc.load_gather(src_n, [idx])
                d = (v >> shift) & 255
                cls = (d << 4) | lane
                pos = plsc.load_gather(hist, [cls])
                plsc.store_scatter(dst_u, [pos], v)
                plsc.store_scatter(dst_n, [pos], nn)
                plsc.addupdate_scatter(hist, [cls], ones)
                return 0
            lax.fori_loop(0, seg, _p, 0)

        with jax.named_scope("p5_sort"):
            _radix_pass(cand_u, cand_n, cand_u2, cand_n2, 0)
            _radix_pass(cand_u2, cand_n2, cand_u, cand_n, 8)
            _radix_pass(cand_u, cand_n, cand_u2, cand_n2, 16)
            _radix_pass(cand_u2, cand_n2, cand_u, cand_n, 24)

        pltpu.sync_copy(cand_u.at[pl.ds(0, 2048)], sh_tv.at[bslot])
        pltpu.sync_copy(cand_n.at[pl.ds(0, 2048)], sh_tn.at[bslot])
    plsc.subcore_barrier()

    r0 = pl.multiple_of(w8 * _RPW, 8)
    pltpu.sync_copy(sh_tv.at[bslot, pl.ds(r0, _RPW)], topv)
    pltpu.sync_copy(sh_tn.at[bslot, pl.ds(r0, _RPW)], topn)

    adesc = []
    for i in range(2):
        adesc.append(pltpu.async_copy(
            anch_hbm.at[topn.at[pl.ds(i * 128, 128)]],
            arow.at[pl.ds(i * 128, 128), :], sem))

    boff = b * 1769472

    def _bases(cc, _):
        n = topn[pl.ds(cc * 16, 16)]
        base = (n % 54) * 32768 + (n // 1728) * 32 + ((n // 54) % 32)
        baseb[pl.ds(cc * 16, 16)] = base + boff
        return 0
    lax.fori_loop(0, 16, _bases, 0, unroll=4)

    def _fill_idx(t, _):
        k = t >> 4
        cc = t & 15
        bv = baseb[pl.ds(cc * 16, 16)]
        idxb[pl.ds(k * _RPW + cc * 16, 16)] = bv + k * 1024
        return 0
    lax.fori_loop(0, 512, _fill_idx, 0, unroll=4)

    def _fire(g):
        for i in range(8):
            jj = g * 8 + i
            pltpu.async_copy(
                bf_hbm.at[idxb.at[pl.ds(jj * 128, 128)]],
                dsoa.at[pl.ds(jj * 128, 128)], gsem)

    def _drain(g):
        for i in range(8):
            jj = g * 8 + i
            pltpu.make_async_copy(
                bf_hbm.at[idxb.at[pl.ds(jj * 128, 128)]],
                dsoa.at[pl.ds(jj * 128, 128)], gsem).wait()

    def _gather_ring(g, _):
        _fire(g)
        @pl.when(g > 0)
        def _():
            _drain(g - 1)
        return 0
    with jax.named_scope("p6_gather"):
        lax.fori_loop(0, 8, _gather_ring, 0)

    for dsc in adesc:
        dsc.wait()

    def _tr(t, _):
        k = t >> 4
        cc = t & 15
        row = cc * 16 + lane
        v = plsc.load_gather(arow, [row, jnp.full((16,), k, i32)])
        asoa[pl.ds(k * _RPW + cc * 16, 16)] = v
        return 0
    with jax.named_scope("p6_transpose"):
        lax.fori_loop(0, 512, _tr, 0, unroll=4)

    with jax.named_scope("p6_drain"):
        _drain(7)

    _imvec = imv[pl.ds(0, 16)]
    wmax = jnp.sum(jnp.where(lane == b * 4 + 1, _imvec, 0.0)) - 1.0
    hmax = jnp.sum(jnp.where(lane == b * 4 + 0, _imvec, 0.0)) - 1.0

    def _transform(cc, _):
        cvec = (cc * 16 + lane) * _OUTW
        for f in range(8):
            q = 4 * f * _RPW + cc * 16
            ax1 = asoa[pl.ds(q, 16)]
            ay1 = asoa[pl.ds(q + _RPW, 16)]
            ax2 = asoa[pl.ds(q + 2 * _RPW, 16)]
            ay2 = asoa[pl.ds(q + 3 * _RPW, 16)]
            dx = dsoa[pl.ds(q, 16)]
            dy = dsoa[pl.ds(q + _RPW, 16)]
            dw = dsoa[pl.ds(q + 2 * _RPW, 16)]
            dh = dsoa[pl.ds(q + 3 * _RPW, 16)]
            ww = ax2 - ax1 + 1.0
            hh = ay2 - ay1 + 1.0
            cx = ax1 + 0.5 * ww
            cy = ay1 + 0.5 * hh
            px = dx * ww + cx
            py = dy * hh + cy
            pw = jnp.exp(dw) * ww
            ph = jnp.exp(dh) * hh
            x1 = jnp.clip(px - 0.5 * pw, 0.0, wmax)
            y1 = jnp.clip(py - 0.5 * ph, 0.0, hmax)
            x2 = jnp.clip(px + 0.5 * pw, 0.0, wmax)
            y2 = jnp.clip(py + 0.5 * ph, 0.0, hmax)
            plsc.store_scatter(outb, [cvec + (1 + 4 * f)], x1)
            plsc.store_scatter(outb, [cvec + (2 + 4 * f)], y1)
            plsc.store_scatter(outb, [cvec + (3 + 4 * f)], x2)
            plsc.store_scatter(outb, [cvec + (4 + 4 * f)], y2)
        return 0
    with jax.named_scope("p6_transform"):
        lax.fori_loop(0, 16, _transform, 0)

    bf32 = b.astype(jnp.float32)

    def _meta_cols(cc, _):
        cvec = (cc * 16 + lane) * _OUTW
        v = topv[pl.ds(cc * 16, 16)]
        key = lax.bitcast_convert_type(~v, u32)
        sb = key ^ ((((~key) >> 31) * u32(0x7FFFFFFF)) + u32(0x80000000))
        plsc.store_scatter(outb, [cvec + 33],
                           lax.bitcast_convert_type(sb, jnp.float32))
        plsc.store_scatter(outb, [cvec],
                           jnp.full((16,), 1.0, jnp.float32) * bf32)
        return 0
    lax.fori_loop(0, 16, _meta_cols, 0)

    pltpu.sync_copy(outb, out_hbm.at[pl.ds(
        pl.multiple_of((b * 2048 + r0) * _OUTW, 8), _RPW * _OUTW)])


@jax.jit
def kernel(scores, bbox_frame, im_info):
    f32 = jnp.float32
    i32 = jnp.int32
    mesh = plsc.VectorSubcoreMesh(core_axis_name="c", subcore_axis_name="s")
    run = functools.partial(
        pl.kernel,
        out_type=jax.ShapeDtypeStruct((_B * 2048 * _OUTW,), f32),
        mesh=mesh,
        compiler_params=pltpu.CompilerParams(needs_layout_passes=False,
                                             use_tc_tiling_on_sc=False),
        scratch_types=[
            pltpu.VMEM((54, 4, 32), f32),
            pltpu.VMEM((_CHUNK,), i32),
            pltpu.VMEM((4096,), i32),
            pltpu.VMEM((256,), i32),
            pltpu.VMEM((256,), i32),
            pltpu.VMEM((16,), i32),
            pltpu.VMEM((8, 16), i32),
            pltpu.VMEM((_CBUF,), i32),
            pltpu.VMEM((_CBUF,), i32),
            pltpu.VMEM((_CBUF,), i32),
            pltpu.VMEM((_CBUF,), i32),
            pltpu.VMEM((_RPW,), i32),
            pltpu.VMEM((_RPW,), i32),
            pltpu.VMEM((_RPW,), i32),
            pltpu.VMEM((32 * _RPW,), i32),
            pltpu.VMEM((32 * _RPW,), f32),
            pltpu.VMEM((_RPW, 32), f32),
            pltpu.VMEM((32 * _RPW,), f32),
            pltpu.VMEM((_RPW * _OUTW,), f32),
            pltpu.VMEM((16,), f32),
            pltpu.VMEM_SHARED((2, 8, 256), i32),
            pltpu.VMEM_SHARED((2, 8, 16), i32),
            pltpu.VMEM_SHARED((2, 16), i32),
            pltpu.VMEM_SHARED((2, 8, _CMAX), i32),
            pltpu.VMEM_SHARED((2, 8, _CMAX), i32),
            pltpu.VMEM_SHARED((2, 2048), i32),
            pltpu.VMEM_SHARED((2, 2048), i32),
            pltpu.SemaphoreType.DMA,
            pltpu.SemaphoreType.DMA,
        ],
    )(_sc_body)

    imf = jnp.pad(im_info, ((0, 0), (0, 1))).reshape(-1)
    out = run(scores, bbox_frame.reshape(-1), jnp.asarray(_ANCHORS), imf)
    return out.reshape(_B, 2048, _OUTW)[:, :_K, :34]
